# baseline (speedup 1.0000x reference)
import functools

import jax
import jax.numpy as jnp
from jax import lax
from jax.experimental import pallas as pl
from jax.experimental.pallas import tpu as pltpu
from jax.experimental.pallas import tpu_sc as plsc

VOCAB = 100000
D = 128
NCLS = 100000
B = 1024
L = 50
LANES = 16
NCHUNK = D // LANES

N_BLK = 2048


def _sc_pool(input_text, emb_table):
    mesh = plsc.VectorSubcoreMesh(core_axis_name="c", subcore_axis_name="s")

    @functools.partial(
        pl.kernel,
        out_type=jax.ShapeDtypeStruct((B, D), jnp.float32),
        mesh=mesh,
        scratch_types=[pltpu.VMEM((L, D), jnp.float32)],
    )
    def sc_kernel(idx_hbm, emb_hbm, out_hbm, rows_ref):
        def body(i_vmem, o_vmem):
            pltpu.sync_copy(emb_hbm.at[i_vmem.at[0]], rows_ref)

            def accum(r, acc):
                return tuple(
                    acc[c] + rows_ref[r, pl.ds(c * LANES, LANES)]
                    for c in range(NCHUNK)
                )

            init = tuple(
                rows_ref[0, pl.ds(c * LANES, LANES)] for c in range(NCHUNK)
            )
            acc = lax.fori_loop(1, L, accum, init)
            for c in range(NCHUNK):
                o_vmem[0, pl.ds(c * LANES, LANES)] = acc[c] * (1.0 / L)

        pltpu.emit_pipeline(
            body,
            grid=(B,),
            in_specs=[pl.BlockSpec((1, L), lambda i: (i, 0))],
            out_specs=[pl.BlockSpec((1, D), lambda i: (i, 0))],
            core_axis_name=("c", "s"),
            dimension_semantics=(pltpu.PARALLEL,),
        )(idx_hbm, out_hbm)

    return sc_kernel(input_text, emb_table)


def _tc_matmul_kernel(p_ref, w_ref, b_ref, o_ref):
    p = p_ref[...].astype(jnp.bfloat16)
    w = w_ref[...].astype(jnp.bfloat16)
    acc = lax.dot_general(
        p, w, (((1,), (1,)), ((), ())), preferred_element_type=jnp.float32
    )
    o_ref[...] = acc + b_ref[...]


def _tc_matmul(pooled, fc_w, fc_b):
    num_n = pl.cdiv(NCLS, N_BLK)
    bias2d = fc_b.reshape(1, NCLS)
    return pl.pallas_call(
        _tc_matmul_kernel,
        grid=(num_n,),
        in_specs=[
            pl.BlockSpec((B, D), lambda n: (0, 0)),
            pl.BlockSpec((N_BLK, D), lambda n: (n, 0)),
            pl.BlockSpec((1, N_BLK), lambda n: (0, n)),
        ],
        out_specs=pl.BlockSpec((B, N_BLK), lambda n: (0, n)),
        out_shape=jax.ShapeDtypeStruct((B, NCLS), jnp.float32),
        compiler_params=pltpu.CompilerParams(
            dimension_semantics=("parallel",),
        ),
    )(pooled, fc_w, bias2d)


def kernel(input_text, emb_table, fc_w, fc_b):
    pooled = _sc_pool(input_text, emb_table)
    return _tc_matmul(pooled, fc_w, fc_b)


# speedup vs baseline: 2.4753x; 2.4753x over previous
import functools

import jax
import jax.numpy as jnp
from jax import lax
from jax.experimental import pallas as pl
from jax.experimental.pallas import tpu as pltpu
from jax.experimental.pallas import tpu_sc as plsc

VOCAB = 100000
D = 128
NCLS = 100000
B = 1024
L = 50
LANES = 16
NCHUNK = D // LANES

N_BLK = 2048


def _sc_pool(input_text, emb_table):
    mesh = plsc.VectorSubcoreMesh(core_axis_name="c", subcore_axis_name="s")

    @functools.partial(
        pl.kernel,
        out_type=jax.ShapeDtypeStruct((B, D), jnp.float32),
        mesh=mesh,
        scratch_types=[pltpu.VMEM((L, D), jnp.float32)],
    )
    def sc_kernel(idx_hbm, emb_hbm, out_hbm, rows_ref):
        def body(i_vmem, o_vmem):
            pltpu.sync_copy(emb_hbm.at[i_vmem.at[0]], rows_ref)

            def accum(r, acc):
                return tuple(
                    acc[c] + rows_ref[r, pl.ds(c * LANES, LANES)]
                    for c in range(NCHUNK)
                )

            init = tuple(
                rows_ref[0, pl.ds(c * LANES, LANES)] for c in range(NCHUNK)
            )
            acc = lax.fori_loop(1, L, accum, init)
            for c in range(NCHUNK):
                o_vmem[0, pl.ds(c * LANES, LANES)] = acc[c] * (1.0 / L)

        pltpu.emit_pipeline(
            body,
            grid=(B,),
            in_specs=[pl.BlockSpec((1, L), lambda i: (i, 0))],
            out_specs=[pl.BlockSpec((1, D), lambda i: (i, 0))],
            core_axis_name=("c", "s"),
            dimension_semantics=(pltpu.PARALLEL,),
        )(idx_hbm, out_hbm)

    return sc_kernel(input_text, emb_table)


def _tc_matmul_kernel(p_ref, w_ref, b_ref, o_ref):
    p = p_ref[...]
    w = w_ref[...]
    acc = lax.dot_general(
        w, p, (((1,), (1,)), ((), ())), preferred_element_type=jnp.float32
    )
    o_ref[...] = acc + b_ref[...]


def _tc_matmul(pooled, fc_w, fc_b):
    num_n = pl.cdiv(NCLS, N_BLK)
    bias2d = fc_b.reshape(NCLS, 1)
    out_t = pl.pallas_call(
        _tc_matmul_kernel,
        grid=(num_n,),
        in_specs=[
            pl.BlockSpec((B, D), lambda n: (0, 0)),
            pl.BlockSpec((N_BLK, D), lambda n: (n, 0)),
            pl.BlockSpec((N_BLK, 1), lambda n: (n, 0)),
        ],
        out_specs=pl.BlockSpec((N_BLK, B), lambda n: (n, 0)),
        out_shape=jax.ShapeDtypeStruct((NCLS, B), jnp.float32),
        compiler_params=pltpu.CompilerParams(
            dimension_semantics=("parallel",),
        ),
    )(pooled, fc_w, bias2d)
    return out_t.T


def kernel(input_text, emb_table, fc_w, fc_b):
    pooled = _sc_pool(input_text, emb_table)
    return _tc_matmul(pooled, fc_w, fc_b)


# speedup vs baseline: 2.7401x; 1.1070x over previous
import functools

import jax
import jax.numpy as jnp
from jax import lax
from jax.experimental import pallas as pl
from jax.experimental.pallas import tpu as pltpu
from jax.experimental.pallas import tpu_sc as plsc

VOCAB = 100000
D = 128
NCLS = 100000
B = 1024
L = 50
LANES = 16
NCHUNK = D // LANES

N_BLK = 2000


def _sc_pool(input_text, emb_table):
    mesh = plsc.VectorSubcoreMesh(core_axis_name="c", subcore_axis_name="s")

    @functools.partial(
        pl.kernel,
        out_type=jax.ShapeDtypeStruct((B, D), jnp.float32),
        mesh=mesh,
        scratch_types=[pltpu.VMEM((L, D), jnp.float32)],
    )
    def sc_kernel(idx_hbm, emb_hbm, out_hbm, rows_ref):
        def body(i_vmem, o_vmem):
            pltpu.sync_copy(emb_hbm.at[i_vmem.at[0]], rows_ref)

            def accum(r, acc):
                return tuple(
                    acc[c] + rows_ref[r, pl.ds(c * LANES, LANES)]
                    for c in range(NCHUNK)
                )

            init = tuple(
                rows_ref[0, pl.ds(c * LANES, LANES)] for c in range(NCHUNK)
            )
            acc = lax.fori_loop(1, L, accum, init)
            for c in range(NCHUNK):
                o_vmem[0, pl.ds(c * LANES, LANES)] = acc[c] * (1.0 / L)

        pltpu.emit_pipeline(
            body,
            grid=(B,),
            in_specs=[pl.BlockSpec((1, L), lambda i: (i, 0))],
            out_specs=[pl.BlockSpec((1, D), lambda i: (i, 0))],
            core_axis_name=("c", "s"),
            dimension_semantics=(pltpu.PARALLEL,),
        )(idx_hbm, out_hbm)

    return sc_kernel(input_text, emb_table)


def _tc_matmul_kernel(p_ref, w_ref, b_ref, o_ref):
    p = p_ref[...]
    w = w_ref[...]
    acc = lax.dot_general(
        w, p, (((1,), (1,)), ((), ())), preferred_element_type=jnp.float32
    )
    ones_row = jnp.ones((1, B), jnp.float32)
    n = pl.program_id(0)
    b_row = b_ref[pl.ds(n, 1), :]
    bias_bc = lax.dot_general(
        b_row, ones_row, (((0,), (0,)), ((), ())),
        preferred_element_type=jnp.float32,
    )
    o_ref[...] = acc + bias_bc


def _tc_matmul(pooled, fc_w, fc_b):
    num_n = NCLS // N_BLK
    bias2d = fc_b.reshape(NCLS // N_BLK, N_BLK)
    out_t = pl.pallas_call(
        _tc_matmul_kernel,
        grid=(num_n,),
        in_specs=[
            pl.BlockSpec((B, D), lambda n: (0, 0)),
            pl.BlockSpec((N_BLK, D), lambda n: (n, 0)),
            pl.BlockSpec((NCLS // N_BLK, N_BLK), lambda n: (0, 0)),
        ],
        out_specs=pl.BlockSpec((N_BLK, B), lambda n: (n, 0)),
        out_shape=jax.ShapeDtypeStruct((NCLS, B), jnp.float32),
        compiler_params=pltpu.CompilerParams(
            dimension_semantics=("parallel",),
        ),
    )(pooled, fc_w, bias2d)
    return out_t.T


def kernel(input_text, emb_table, fc_w, fc_b):
    pooled = _sc_pool(input_text, emb_table)
    return _tc_matmul(pooled, fc_w, fc_b)


# speedup vs baseline: 2.8722x; 1.0482x over previous
import functools

import jax
import jax.numpy as jnp
from jax import lax
from jax.experimental import pallas as pl
from jax.experimental.pallas import tpu as pltpu
from jax.experimental.pallas import tpu_sc as plsc

VOCAB = 100000
D = 128
NCLS = 100000
B = 1024
L = 50
LANES = 16
NCHUNK = D // LANES

N_BLK = 2000


NSUB = 32
ROWS = B // NSUB


def _sc_pool(input_text, emb_table):
    mesh = plsc.VectorSubcoreMesh(core_axis_name="c", subcore_axis_name="s")

    @functools.partial(
        pl.kernel,
        out_type=jax.ShapeDtypeStruct((B, D), jnp.float32),
        mesh=mesh,
        scratch_types=[
            pltpu.VMEM((ROWS, L), jnp.int32),
            pltpu.VMEM((L, D), jnp.float32),
            pltpu.VMEM((L, D), jnp.float32),
            pltpu.VMEM((ROWS, D), jnp.float32),
            pltpu.SemaphoreType.DMA,
            pltpu.SemaphoreType.DMA,
        ],
    )
    def sc_kernel(idx_hbm, emb_hbm, out_hbm, idx_v, buf0, buf1, pooled_v,
                  sem0, sem1):
        wid = lax.axis_index("s") * 2 + lax.axis_index("c")
        base = wid * ROWS
        pltpu.sync_copy(idx_hbm.at[pl.ds(base, ROWS)], idx_v)
        pltpu.async_copy(emb_hbm.at[idx_v.at[0]], buf0, sem0)
        pltpu.async_copy(emb_hbm.at[idx_v.at[1]], buf1, sem1)

        def accum_store(b, buf):
            accs = [buf[0, pl.ds(c * LANES, LANES)] for c in range(NCHUNK)]
            for r in range(1, L):
                for c in range(NCHUNK):
                    accs[c] = accs[c] + buf[r, pl.ds(c * LANES, LANES)]
            for c in range(NCHUNK):
                pooled_v[b, pl.ds(c * LANES, LANES)] = accs[c] * (1.0 / L)

        @pl.loop(0, ROWS, step=2)
        def _(b):
            pltpu.make_async_copy(emb_hbm.at[idx_v.at[0]], buf0, sem0).wait()
            accum_store(b, buf0)

            @pl.when(b + 2 < ROWS)
            def _():
                pltpu.async_copy(emb_hbm.at[idx_v.at[b + 2]], buf0, sem0)

            pltpu.make_async_copy(emb_hbm.at[idx_v.at[1]], buf1, sem1).wait()
            accum_store(b + 1, buf1)

            @pl.when(b + 3 < ROWS)
            def _():
                pltpu.async_copy(emb_hbm.at[idx_v.at[b + 3]], buf1, sem1)

        pltpu.sync_copy(pooled_v, out_hbm.at[pl.ds(base, ROWS)])

    return sc_kernel(input_text, emb_table)


def _tc_matmul_kernel(p_ref, w_ref, b_ref, o_ref):
    p = p_ref[...]
    w = w_ref[...]
    acc = lax.dot_general(
        w, p, (((1,), (1,)), ((), ())), preferred_element_type=jnp.float32
    )
    ones_row = jnp.ones((1, B), jnp.float32)
    n = pl.program_id(0)
    b_row = b_ref[pl.ds(n, 1), :]
    bias_bc = lax.dot_general(
        b_row, ones_row, (((0,), (0,)), ((), ())),
        preferred_element_type=jnp.float32,
    )
    o_ref[...] = acc + bias_bc


def _tc_matmul(pooled, fc_w, fc_b):
    num_n = NCLS // N_BLK
    bias2d = fc_b.reshape(NCLS // N_BLK, N_BLK)
    out_t = pl.pallas_call(
        _tc_matmul_kernel,
        grid=(num_n,),
        in_specs=[
            pl.BlockSpec((B, D), lambda n: (0, 0)),
            pl.BlockSpec((N_BLK, D), lambda n: (n, 0)),
            pl.BlockSpec((NCLS // N_BLK, N_BLK), lambda n: (0, 0)),
        ],
        out_specs=pl.BlockSpec((N_BLK, B), lambda n: (n, 0)),
        out_shape=jax.ShapeDtypeStruct((NCLS, B), jnp.float32),
        compiler_params=pltpu.CompilerParams(
            dimension_semantics=("parallel",),
        ),
    )(pooled, fc_w, bias2d)
    return out_t.T


def kernel(input_text, emb_table, fc_w, fc_b):
    pooled = _sc_pool(input_text, emb_table)
    return _tc_matmul(pooled, fc_w, fc_b)
